# baseline (speedup 1.0000x reference)
import functools

import jax
import jax.numpy as jnp
from jax import lax
from jax.experimental import pallas as pl
from jax.experimental.pallas import tpu as pltpu
from jax.experimental.pallas import tpu_sc as plsc

D = 64
S = 200
NC = 2
NS = 16
NW = NC * NS
G1 = 128
G2 = S - G1
LANES = 16


def _pool_body(tok_hbm, table_hbm, out_hbm, idx_v, gbuf, pooled, sem):
    cid = lax.axis_index("c")
    sid = lax.axis_index("s")
    wid = cid * NS + sid
    rows = pooled.shape[0]
    base = wid * rows
    pltpu.sync_copy(tok_hbm.at[pl.ds(base * S, rows * S)], idx_v)

    inv = jnp.float32(1.0 / S)

    def row_body(r, carry):
        o = pl.multiple_of(r * S, 8)
        c1 = pltpu.async_copy(table_hbm.at[idx_v.at[pl.ds(o, G1)]],
                              gbuf.at[pl.ds(0, G1)], sem)
        c2 = pltpu.async_copy(table_hbm.at[idx_v.at[pl.ds(o + G1, G2)]],
                              gbuf.at[pl.ds(G1, G2)], sem)
        c1.wait()
        c2.wait()

        def acc_body(i, accs):
            a0, a1, a2, a3 = accs
            for u in range(4):
                t = i * 4 + u
                a0 = a0 + gbuf[t, pl.ds(0 * LANES, LANES)]
                a1 = a1 + gbuf[t, pl.ds(1 * LANES, LANES)]
                a2 = a2 + gbuf[t, pl.ds(2 * LANES, LANES)]
                a3 = a3 + gbuf[t, pl.ds(3 * LANES, LANES)]
            return (a0, a1, a2, a3)

        z = jnp.zeros((LANES,), jnp.float32)
        a0, a1, a2, a3 = lax.fori_loop(0, S // 4, acc_body, (z, z, z, z))
        pooled[r, pl.ds(0 * LANES, LANES)] = a0 * inv
        pooled[r, pl.ds(1 * LANES, LANES)] = a1 * inv
        pooled[r, pl.ds(2 * LANES, LANES)] = a2 * inv
        pooled[r, pl.ds(3 * LANES, LANES)] = a3 * inv
        return carry

    lax.fori_loop(0, rows, row_body, 0)
    pltpu.sync_copy(pooled, out_hbm.at[pl.ds(base, rows)])


def _pool(tokens_flat, table):
    batch = tokens_flat.shape[0] // S
    rows = batch // NW
    mesh = plsc.VectorSubcoreMesh(core_axis_name="c", subcore_axis_name="s")
    f = pl.kernel(
        _pool_body,
        mesh=mesh,
        compiler_params=pltpu.CompilerParams(use_tc_tiling_on_sc=False),
        out_type=jax.ShapeDtypeStruct((batch, D), jnp.float32),
        scratch_types=[
            pltpu.VMEM((rows * S,), jnp.int32),
            pltpu.VMEM((S, D), jnp.float32),
            pltpu.VMEM((rows, D), jnp.float32),
            pltpu.SemaphoreType.DMA,
        ],
    )
    return f(tokens_flat, table)


def _mm_body(x_ref, w_ref, b_ref, o_ref):
    o_ref[...] = lax.dot_general(
        x_ref[...], w_ref[...], (((1,), (1,)), ((), ())),
        preferred_element_type=jnp.float32) + b_ref[...]


def _classify(pooled, W, b):
    batch = pooled.shape[0]
    ncls = W.shape[0]
    return pl.pallas_call(
        _mm_body,
        out_shape=jax.ShapeDtypeStruct((batch, ncls), jnp.float32),
    )(pooled, W, b.reshape(1, ncls))


def kernel(utteranceTokens, embedding_table, W, b):
    batch = utteranceTokens.shape[0]
    pooled = _pool(utteranceTokens.reshape(batch * S), embedding_table)
    return _classify(pooled, W, b)


# speedup vs baseline: 1.4925x; 1.4925x over previous
import functools

import jax
import jax.numpy as jnp
from jax import lax
from jax.experimental import pallas as pl
from jax.experimental.pallas import tpu as pltpu
from jax.experimental.pallas import tpu_sc as plsc

D = 64
C = 128
S = 200
NC = 2
NS = 16
NW = NC * NS
G1 = 128
G2 = S - G1
LANES = 16
VB = 4096


def _project_body(et_ref, w_ref, b_ref, o_ref):
    o_ref[...] = lax.dot_general(
        et_ref[...], w_ref[...], (((0,), (1,)), ((), ())),
        preferred_element_type=jnp.float32) + b_ref[...]


def _project(table_t, W, b):
    vocab = table_t.shape[1]
    nblk = pl.cdiv(vocab, VB)
    return pl.pallas_call(
        _project_body,
        grid=(nblk,),
        in_specs=[
            pl.BlockSpec((D, VB), lambda i: (0, i)),
            pl.BlockSpec((C, D), lambda i: (0, 0)),
            pl.BlockSpec((1, C), lambda i: (0, 0)),
        ],
        out_specs=pl.BlockSpec((VB, C), lambda i: (i, 0)),
        out_shape=jax.ShapeDtypeStruct((vocab, C), jnp.float32),
    )(table_t, W, b.reshape(1, C))


def _pool_body(tok_hbm, p_hbm, out_hbm, idx_v, bufa, bufb, pooled, sema, semb):
    cid = lax.axis_index("c")
    sid = lax.axis_index("s")
    wid = cid * NS + sid
    rows = pooled.shape[0]
    base = wid * rows
    pltpu.sync_copy(tok_hbm.at[pl.ds(base * S, rows * S)], idx_v)

    inv = jnp.float32(1.0 / S)

    def issue(r, buf, sem):
        o = pl.multiple_of(r * S, 8)
        pltpu.async_copy(p_hbm.at[idx_v.at[pl.ds(o, G1)]],
                         buf.at[pl.ds(0, G1)], sem)
        pltpu.async_copy(p_hbm.at[idx_v.at[pl.ds(o + G1, G2)]],
                         buf.at[pl.ds(G1, G2)], sem)

    def drain(r, buf, sem):
        o = pl.multiple_of(r * S, 8)
        pltpu.make_async_copy(p_hbm.at[idx_v.at[pl.ds(o, G1)]],
                              buf.at[pl.ds(0, G1)], sem).wait()
        pltpu.make_async_copy(p_hbm.at[idx_v.at[pl.ds(o + G1, G2)]],
                              buf.at[pl.ds(G1, G2)], sem).wait()

    def accum(r, buf):
        def acc_body(i, accs):
            new = []
            for u in range(2):
                t = i * 2 + u
                new = [a + buf[t, pl.ds(j * LANES, LANES)]
                       for j, a in enumerate(accs if u == 0 else new)]
            return tuple(new)

        z = jnp.zeros((LANES,), jnp.float32)
        accs = lax.fori_loop(0, S // 2, acc_body, (z,) * (C // LANES))
        for j, a in enumerate(accs):
            pooled[r, pl.ds(j * LANES, LANES)] = a * inv

    issue(0, bufa, sema)

    def pair_body(rr, carry):
        r0 = rr * 2
        issue(r0 + 1, bufb, semb)
        drain(r0, bufa, sema)
        accum(r0, bufa)

        @pl.when(rr < rows // 2 - 1)
        def _():
            issue(r0 + 2, bufa, sema)

        drain(r0 + 1, bufb, semb)
        accum(r0 + 1, bufb)
        return carry

    lax.fori_loop(0, rows // 2, pair_body, 0)
    pltpu.sync_copy(pooled, out_hbm.at[pl.ds(base, rows)])


def _pool(tokens_flat, P):
    batch = tokens_flat.shape[0] // S
    rows = batch // NW
    mesh = plsc.VectorSubcoreMesh(core_axis_name="c", subcore_axis_name="s")
    f = pl.kernel(
        _pool_body,
        mesh=mesh,
        compiler_params=pltpu.CompilerParams(use_tc_tiling_on_sc=True),
        out_type=jax.ShapeDtypeStruct((batch, C), jnp.float32),
        scratch_types=[
            pltpu.VMEM((rows * S,), jnp.int32),
            pltpu.VMEM((S, C), jnp.float32),
            pltpu.VMEM((S, C), jnp.float32),
            pltpu.VMEM((rows, C), jnp.float32),
            pltpu.SemaphoreType.DMA,
            pltpu.SemaphoreType.DMA,
        ],
    )
    return f(tokens_flat, P)


def kernel(utteranceTokens, embedding_table, W, b):
    batch = utteranceTokens.shape[0]
    P = _project(embedding_table.T, W, b)
    return _pool(utteranceTokens.reshape(batch * S), P)
